# Initial kernel scaffold; baseline (speedup 1.0000x reference)
import jax
import jax.numpy as jnp
from jax.experimental import pallas as pl


def kernel(edge_index, edge_type, tensor_slice, W1, root1, b1, W2, root2, b2):
    raise NotImplementedError("write your pallas kernel here")


# speedup vs baseline: 4.3263x; 4.3263x over previous
import functools

import jax
import jax.numpy as jnp
from jax import lax
from jax.experimental import pallas as pl
from jax.experimental.pallas import tpu as pltpu
from jax.experimental.pallas import tpu_sc as plsc

N = 10000
E = 640000
R = 90
H = 64
C = 4
C2 = 8

NC = 2
NS = 16
NW = NC * NS
EPW = E // NW
CK = 80
NCH = EPW // CK
NP = 10240
NPT = NP // NS
ST = 128


def _make_sc_layer(D):
    mesh = plsc.VectorSubcoreMesh(core_axis_name="c", subcore_axis_name="s")

    @functools.partial(
        pl.kernel,
        mesh=mesh,
        compiler_params=pltpu.CompilerParams(use_tc_tiling_on_sc=False),
        out_type=jax.ShapeDtypeStruct((NC, NP, D), jnp.float32),
        scratch_types=[
            pltpu.VMEM((NCH, CK), jnp.int32),
            pltpu.VMEM((NCH, CK), jnp.int32),
            pltpu.VMEM((2, CK, D), jnp.float32),
            pltpu.VMEM((ST, D), jnp.float32),
            pltpu.VMEM_SHARED((NP, D), jnp.float32),
            pltpu.SemaphoreType.DMA,
            pltpu.SemaphoreType.DMA,
        ],
    )
    def sc_layer(table_h, gidx_h, didx_h, zer_h, out_h,
                 gi_v, di_v, rows_v, stage_v, acc_s, sem0, sem1):
        c = lax.axis_index("c")
        s = lax.axis_index("s")
        w = c * NS + s

        pltpu.sync_copy(zer_h, stage_v)
        for t in range(NPT // ST):
            pltpu.sync_copy(stage_v, acc_s.at[pl.ds(s * NPT + t * ST, ST)])
        pltpu.sync_copy(gidx_h.at[w], gi_v)
        pltpu.sync_copy(didx_h.at[w], di_v)
        plsc.subcore_barrier()

        cp0 = pltpu.async_copy(table_h.at[gi_v.at[0]], rows_v.at[0], sem0)
        cp1 = pltpu.async_copy(table_h.at[gi_v.at[1]], rows_v.at[1], sem1)

        @pl.loop(0, NCH, step=2)
        def _(j):
            pltpu.make_async_copy(table_h.at[gi_v.at[j]], rows_v.at[0],
                                  sem0).wait()
            pltpu.sync_copy(rows_v.at[0], acc_s.at[di_v.at[j]], add=True)

            @pl.when(j + 2 < NCH)
            def _():
                pltpu.async_copy(table_h.at[gi_v.at[j + 2]], rows_v.at[0],
                                 sem0)

            pltpu.make_async_copy(table_h.at[gi_v.at[j + 1]], rows_v.at[1],
                                  sem1).wait()
            pltpu.sync_copy(rows_v.at[1], acc_s.at[di_v.at[j + 1]], add=True)

            @pl.when(j + 3 < NCH)
            def _():
                pltpu.async_copy(table_h.at[gi_v.at[j + 3]], rows_v.at[1],
                                 sem1)

        plsc.subcore_barrier()

        for t in range(NPT // ST):
            pltpu.sync_copy(acc_s.at[pl.ds(s * NPT + t * ST, ST)], stage_v)
            pltpu.sync_copy(stage_v, out_h.at[c, pl.ds(s * NPT + t * ST, ST)])

    return sc_layer


def _tc_dense(p, root1, b1, w2rs, root2):
    BN = 1000

    def body(p_ref, r1_ref, b1_ref, w_ref, rt_ref, h_ref, x2_ref):
        x1 = jnp.maximum(p_ref[0] + p_ref[1] + r1_ref[...] + b1_ref[...], 0.0)
        h_ref[...] = jnp.dot(x1, w_ref[...], preferred_element_type=jnp.float32)
        x2_ref[...] = jnp.dot(x1, rt_ref[...], preferred_element_type=jnp.float32)

    return pl.pallas_call(
        body,
        grid=(N // BN,),
        in_specs=[
            pl.BlockSpec((NC, BN, H), lambda i: (0, i, 0)),
            pl.BlockSpec((BN, H), lambda i: (i, 0)),
            pl.BlockSpec((1, H), lambda i: (0, 0)),
            pl.BlockSpec((H, R * C2), lambda i: (0, 0)),
            pl.BlockSpec((H, C), lambda i: (0, 0)),
        ],
        out_specs=[
            pl.BlockSpec((BN, R * C2), lambda i: (i, 0)),
            pl.BlockSpec((BN, C), lambda i: (i, 0)),
        ],
        out_shape=[
            jax.ShapeDtypeStruct((N, R * C2), jnp.float32),
            jax.ShapeDtypeStruct((N, C), jnp.float32),
        ],
    )(p, root1, b1, w2rs, root2)


def _tc_final(p2, xr2, b2):
    def body(p_ref, x_ref, b_ref, o_ref):
        agg = (p_ref[0] + p_ref[1])[:, :C]
        t = agg + x_ref[...] + b_ref[...]
        m = jnp.max(t, axis=1, keepdims=True)
        lse = jnp.log(jnp.sum(jnp.exp(t - m), axis=1, keepdims=True)) + m
        o_ref[...] = t - lse

    return pl.pallas_call(
        body,
        grid=(1,),
        in_specs=[
            pl.BlockSpec((NC, N, C2), lambda i: (0, 0, 0)),
            pl.BlockSpec((N, C), lambda i: (0, 0)),
            pl.BlockSpec((1, C), lambda i: (0, 0)),
        ],
        out_specs=pl.BlockSpec((N, C), lambda i: (0, 0)),
        out_shape=jax.ShapeDtypeStruct((N, C), jnp.float32),
    )(p2, xr2, b2)


def kernel(edge_index, edge_type, tensor_slice, W1, root1, b1, W2, root2, b2):
    src = edge_index[0]
    dst = edge_index[1]

    gidx1 = (edge_type * N + src).reshape(NW, NCH, CK)
    gidx2 = (src * R + edge_type).reshape(NW, NCH, CK)
    didx = dst.reshape(NW, NCH, CK)

    zeros1 = jnp.zeros((ST, H), jnp.float32)
    zeros2 = jnp.zeros((ST, C2), jnp.float32)

    p1 = _make_sc_layer(H)(W1.reshape(R * N, H), gidx1, didx, zeros1)

    w2rs = jnp.pad(W2.transpose(1, 0, 2),
                   ((0, 0), (0, 0), (0, C2 - C))).reshape(H, R * C2)
    h, xr2 = _tc_dense(p1, root1, b1.reshape(1, H), w2rs, root2)

    p2 = _make_sc_layer(C2)(h.reshape(N * R, C2), gidx2, didx, zeros2)

    return _tc_final(p2, xr2, b2.reshape(1, C))
